# baseline (speedup 1.0000x reference)
import functools

import jax
import jax.numpy as jnp
from jax import lax
from jax.experimental import pallas as pl
from jax.experimental.pallas import tpu as pltpu
from jax.experimental.pallas import tpu_sc as plsc

VOCAB = 100000
D = 128
SEQ = 200
BATCH = 1024
TOKENS = BATCH * SEQ
EPS = 1e-12

_INFO = plsc.get_sparse_core_info()
_NC = _INFO.num_cores
_NS = _INFO.num_subcores
_NW = _NC * _NS
_PER_W = TOKENS // _NW
_CH = 256
_NIT = _PER_W // _CH
_NG = _CH // 16


def _newton_rsqrt_scalar(v):
    i = lax.bitcast_convert_type(v, jnp.int32)
    y = lax.bitcast_convert_type(
        jnp.int32(0x5F3759DF) - lax.shift_right_arithmetic(i, jnp.int32(1)),
        jnp.float32)
    for _ in range(3):
        y = y * (1.5 - 0.5 * v * y * y)
    return y


def _sc_fused(idx_flat, tt_flat, word_emb, pos_emb, type_emb, gamma, beta):
    mesh = plsc.VectorSubcoreMesh(core_axis_name="c", subcore_axis_name="s")

    @functools.partial(
        pl.kernel,
        mesh=mesh,
        compiler_params=pltpu.CompilerParams(needs_layout_passes=False),
        out_type=jax.ShapeDtypeStruct((TOKENS, D), jnp.float32),
        scratch_types=[
            pltpu.VMEM((_CH,), jnp.int32),
            pltpu.VMEM((_CH,), jnp.int32),
            pltpu.VMEM((_CH, D), jnp.float32),
            pltpu.VMEM((2 * SEQ, D), jnp.float32),
            pltpu.VMEM((2, D), jnp.float32),
            pltpu.VMEM((D,), jnp.float32),
            pltpu.VMEM((D,), jnp.float32),
            pltpu.SemaphoreType.DMA,
        ],
    )
    def k(idx_hbm, tt_hbm, table_hbm, pos_hbm, type_hbm, gamma_hbm, beta_hbm,
          out_hbm, idx_v, tt_v, rows_v, pt_v, ty_v, g_v, b_v, sem):
        wid = lax.axis_index("s") * _NC + lax.axis_index("c")
        base = wid * _PER_W

        pltpu.sync_copy(pos_hbm.at[pl.ds(0, SEQ)], pt_v.at[pl.ds(0, SEQ)])
        pltpu.sync_copy(pos_hbm.at[pl.ds(0, SEQ)], pt_v.at[pl.ds(SEQ, SEQ)])
        pltpu.sync_copy(type_hbm, ty_v)
        pltpu.sync_copy(gamma_hbm, g_v)
        pltpu.sync_copy(beta_hbm, b_v)

        def ptbody(p, c):
            for j in range(8):
                sl = pl.ds(16 * j, 16)
                pt_v[p, sl] = pt_v[p, sl] + ty_v[0, sl]
                pt_v[SEQ + p, sl] = pt_v[SEQ + p, sl] + ty_v[1, sl]
            return c

        lax.fori_loop(0, SEQ, ptbody, 0)

        gm = [g_v[pl.ds(16 * j, 16)] for j in range(8)]
        bt = [b_v[pl.ds(16 * j, 16)] for j in range(8)]

        def chunk(i, carry):
            off = base + i * _CH
            pltpu.sync_copy(idx_hbm.at[pl.ds(off, _CH)], idx_v)
            pltpu.sync_copy(tt_hbm.at[pl.ds(off, _CH)], tt_v)
            pltpu.async_copy(table_hbm.at[idx_v], rows_v, sem).wait()

            def group(g, c):
                gbase = g * 16
                ttv = tt_v[pl.ds(gbase, 16)]
                for t in range(16):
                    rowt = gbase + t
                    tt_t = jnp.squeeze(lax.slice(ttv, (t,), (t + 1,)))
                    pos_t = jnp.remainder(off + rowt, SEQ)
                    r_t = tt_t * SEQ + pos_t
                    xs = []
                    for j in range(8):
                        sl = pl.ds(16 * j, 16)
                        xs.append(rows_v[rowt, sl] + pt_v[r_t, sl])
                    s = xs[0]
                    for j in range(1, 8):
                        s = s + xs[j]
                    q = xs[0] * xs[0]
                    for j in range(1, 8):
                        q = q + xs[j] * xs[j]
                    ssum = jnp.sum(s)
                    qsum = jnp.sum(q)
                    mean = ssum * (1.0 / D)
                    var = qsum * (1.0 / D) - mean * mean
                    rstd = _newton_rsqrt_scalar(var + EPS)
                    for j in range(8):
                        sl = pl.ds(16 * j, 16)
                        rows_v[rowt, sl] = (xs[j] - mean) * rstd * gm[j] + bt[j]
                return c

            lax.fori_loop(0, _NG, group, 0)
            pltpu.sync_copy(rows_v, out_hbm.at[pl.ds(off, _CH)])
            return carry

        lax.fori_loop(0, _NIT, chunk, 0)

    return k(idx_flat, tt_flat, word_emb, pos_emb, type_emb, gamma, beta)


def kernel(input_ids, token_type_ids, word_emb, pos_emb, type_emb, gamma, beta):
    idx_flat = input_ids.reshape(TOKENS).astype(jnp.int32)
    tt_flat = token_type_ids.reshape(TOKENS).astype(jnp.int32)
    out = _sc_fused(idx_flat, tt_flat, word_emb, pos_emb, type_emb, gamma, beta)
    return out.reshape(BATCH, SEQ, D)


# speedup vs baseline: 2.6199x; 2.6199x over previous
import functools

import jax
import jax.numpy as jnp
from jax import lax
from jax.experimental import pallas as pl
from jax.experimental.pallas import tpu as pltpu
from jax.experimental.pallas import tpu_sc as plsc

VOCAB = 100000
D = 128
SEQ = 200
BATCH = 1024
TOKENS = BATCH * SEQ
EPS = 1e-12

_INFO = plsc.get_sparse_core_info()
_NC = _INFO.num_cores
_NS = _INFO.num_subcores
_NW = _NC * _NS
_PER_W = TOKENS // _NW
_CHUNK = 400
_NITER = _PER_W // _CHUNK


def _sc_gather(idx_flat, table):
    mesh = plsc.VectorSubcoreMesh(core_axis_name="c", subcore_axis_name="s")

    @functools.partial(
        pl.kernel,
        mesh=mesh,
        out_type=jax.ShapeDtypeStruct((TOKENS, D), jnp.float32),
        scratch_types=[
            pltpu.VMEM((_CHUNK,), jnp.int32),
            pltpu.VMEM((_CHUNK, D), jnp.float32),
            pltpu.SemaphoreType.DMA,
        ],
    )
    def k(idx_hbm, table_hbm, out_hbm, idx_v, rows_v, sem):
        wid = lax.axis_index("s") * _NC + lax.axis_index("c")
        base = wid * _PER_W

        def body(i, carry):
            off = base + i * _CHUNK
            pltpu.sync_copy(idx_hbm.at[pl.ds(off, _CHUNK)], idx_v)
            pltpu.async_copy(table_hbm.at[idx_v], rows_v, sem).wait()
            pltpu.sync_copy(rows_v, out_hbm.at[pl.ds(off, _CHUNK)])
            return carry

        lax.fori_loop(0, _NITER, body, 0)

    return k(idx_flat, table)


def _tc_body(rows_ref, tt_ref, pos_ref, type_ref, gamma_ref, beta_ref, out_ref):
    x = rows_ref[...]
    tt = tt_ref[...]
    pos = pos_ref[...]
    t0 = type_ref[0, :]
    t1 = type_ref[1, :]
    te = jnp.where((tt[..., None] == 0), t0[None, None, :], t1[None, None, :])
    x = x + pos[None, :, :] + te
    mean = jnp.mean(x, axis=-1, keepdims=True)
    var = jnp.mean(jnp.square(x - mean), axis=-1, keepdims=True)
    y = (x - mean) * lax.rsqrt(var + EPS)
    out_ref[...] = y * gamma_ref[...] + beta_ref[...]


def _tc_add_ln(rows, token_type_ids, pos_emb, type_emb, gamma, beta):
    BB = 64
    grid = (BATCH // BB,)
    return pl.pallas_call(
        _tc_body,
        grid=grid,
        in_specs=[
            pl.BlockSpec((BB, SEQ, D), lambda i: (i, 0, 0)),
            pl.BlockSpec((BB, SEQ), lambda i: (i, 0)),
            pl.BlockSpec((SEQ, D), lambda i: (0, 0)),
            pl.BlockSpec((2, D), lambda i: (0, 0)),
            pl.BlockSpec((D,), lambda i: (0,)),
            pl.BlockSpec((D,), lambda i: (0,)),
        ],
        out_specs=pl.BlockSpec((BB, SEQ, D), lambda i: (i, 0, 0)),
        out_shape=jax.ShapeDtypeStruct((BATCH, SEQ, D), jnp.float32),
    )(rows, token_type_ids, pos_emb, type_emb, gamma, beta)


def kernel(input_ids, token_type_ids, word_emb, pos_emb, type_emb, gamma, beta):
    idx_flat = input_ids.reshape(TOKENS).astype(jnp.int32)
    rows = _sc_gather(idx_flat, word_emb)
    rows = rows.reshape(BATCH, SEQ, D)
    tt = token_type_ids.astype(jnp.int32)
    pos = pos_emb[:SEQ]
    return _tc_add_ln(rows, tt, pos, type_emb, gamma, beta)


# speedup vs baseline: 2.8116x; 1.0731x over previous
import functools

import jax
import jax.numpy as jnp
from jax import lax
from jax.experimental import pallas as pl
from jax.experimental.pallas import tpu as pltpu
from jax.experimental.pallas import tpu_sc as plsc

VOCAB = 100000
D = 128
SEQ = 200
BATCH = 1024
TOKENS = BATCH * SEQ
EPS = 1e-12

_INFO = plsc.get_sparse_core_info()
_NC = _INFO.num_cores
_NS = _INFO.num_subcores
_NW = _NC * _NS
_PER_W = TOKENS // _NW
_CHUNK = 400
_NITER = _PER_W // _CHUNK


def _sc_gather(idx_flat, table):
    mesh = plsc.VectorSubcoreMesh(core_axis_name="c", subcore_axis_name="s")

    @functools.partial(
        pl.kernel,
        mesh=mesh,
        out_type=jax.ShapeDtypeStruct((TOKENS, D), jnp.float32),
        scratch_types=[
            pltpu.VMEM((_CHUNK,), jnp.int32),
            pltpu.VMEM((_CHUNK,), jnp.int32),
            pltpu.VMEM((_CHUNK, D), jnp.float32),
            pltpu.VMEM((_CHUNK, D), jnp.float32),
            pltpu.SemaphoreType.DMA,
            pltpu.SemaphoreType.DMA,
            pltpu.SemaphoreType.DMA,
            pltpu.SemaphoreType.DMA,
        ],
    )
    def k(idx_hbm, table_hbm, out_hbm, idx0, idx1, rows0, rows1,
          gs0, gs1, ws0, ws1):
        wid = lax.axis_index("s") * _NC + lax.axis_index("c")
        base = wid * _PER_W

        idxs = [idx0, idx1]
        bufs = [rows0, rows1]
        gsems = [gs0, gs1]
        wsems = [ws0, ws1]
        g = [None, None]
        w = [None] * _NITER

        pltpu.sync_copy(idx_hbm.at[pl.ds(base, _CHUNK)], idxs[0])
        g[0] = pltpu.async_copy(table_hbm.at[idxs[0]], bufs[0], gsems[0])

        for i in range(_NITER):
            cur = i % 2
            if i + 1 < _NITER:
                nxt = (i + 1) % 2
                if i >= 1:
                    w[i - 1].wait()
                off_n = base + (i + 1) * _CHUNK
                pltpu.sync_copy(idx_hbm.at[pl.ds(off_n, _CHUNK)], idxs[nxt])
                g[nxt] = pltpu.async_copy(
                    table_hbm.at[idxs[nxt]], bufs[nxt], gsems[nxt])
            g[cur].wait()
            off = base + i * _CHUNK
            w[i] = pltpu.async_copy(
                bufs[cur], out_hbm.at[pl.ds(off, _CHUNK)], wsems[cur])

        w[_NITER - 2].wait()
        w[_NITER - 1].wait()

    return k(idx_flat, table)


def _tc_body(rows_ref, tt_ref, pos_ref, type_ref, gamma_ref, beta_ref, out_ref):
    x = rows_ref[...]
    tt = tt_ref[...]
    pos = pos_ref[...]
    t0 = type_ref[0, :]
    t1 = type_ref[1, :]
    te = jnp.where((tt[..., None] == 0), t0[None, None, :], t1[None, None, :])
    x = x + pos[None, :, :] + te
    mean = jnp.mean(x, axis=-1, keepdims=True)
    var = jnp.mean(jnp.square(x - mean), axis=-1, keepdims=True)
    y = (x - mean) * lax.rsqrt(var + EPS)
    out_ref[...] = y * gamma_ref[...] + beta_ref[...]


def _tc_add_ln(rows, token_type_ids, pos_emb, type_emb, gamma, beta):
    BB = 64
    grid = (BATCH // BB,)
    return pl.pallas_call(
        _tc_body,
        grid=grid,
        in_specs=[
            pl.BlockSpec((BB, SEQ, D), lambda i: (i, 0, 0)),
            pl.BlockSpec((BB, SEQ), lambda i: (i, 0)),
            pl.BlockSpec((SEQ, D), lambda i: (0, 0)),
            pl.BlockSpec((2, D), lambda i: (0, 0)),
            pl.BlockSpec((D,), lambda i: (0,)),
            pl.BlockSpec((D,), lambda i: (0,)),
        ],
        out_specs=pl.BlockSpec((BB, SEQ, D), lambda i: (i, 0, 0)),
        out_shape=jax.ShapeDtypeStruct((BATCH, SEQ, D), jnp.float32),
    )(rows, token_type_ids, pos_emb, type_emb, gamma, beta)


def kernel(input_ids, token_type_ids, word_emb, pos_emb, type_emb, gamma, beta):
    idx_flat = input_ids.reshape(TOKENS).astype(jnp.int32)
    rows = _sc_gather(idx_flat, word_emb)
    rows = rows.reshape(BATCH, SEQ, D)
    tt = token_type_ids.astype(jnp.int32)
    pos = pos_emb[:SEQ]
    return _tc_add_ln(rows, tt, pos, type_emb, gamma, beta)
